# baseline (speedup 1.0000x reference)
import functools

import jax
import jax.numpy as jnp
from jax.experimental import pallas as pl

N = 8192
HIDDEN = 32
REG = 1.0
CAPACITY = 3
BLK = 256


def _scores_kernel(x_ref, w1_ref, b1_ref, w2_ref, b2_ref, s_ref):
    x = x_ref[:, :]
    h = jnp.maximum(x * w1_ref[0, :][None, :] + b1_ref[0, :][None, :], 0.0)
    s = h @ w2_ref[:, :] + b2_ref[0, 0]
    s_ref[:, :] = s.reshape(1, N)


def _softrank_kernel(srow_ref, sfull_ref, out_ref):
    srow = srow_ref[0, :]
    sfull = sfull_ref[0, :]
    diff = (srow[:, None] - sfull[None, :]) * (1.0 / REG)
    p = jax.nn.sigmoid(diff)
    ranks = jnp.sum(p, axis=1) + 0.5
    other = ranks % CAPACITY
    out = (ranks - other) / CAPACITY + 1.0
    out_ref[:, :] = out.reshape(BLK, 1)


@jax.jit
def kernel(x, W1, b1, W2, b2):
    s = pl.pallas_call(
        _scores_kernel,
        out_shape=jax.ShapeDtypeStruct((1, N), jnp.float32),
    )(x, W1, b1.reshape(1, HIDDEN), W2, b2.reshape(1, 1))

    out = pl.pallas_call(
        _softrank_kernel,
        grid=(N // BLK,),
        in_specs=[
            pl.BlockSpec((1, BLK), lambda i: (0, i)),
            pl.BlockSpec((1, N), lambda i: (0, 0)),
        ],
        out_specs=pl.BlockSpec((BLK, 1), lambda i: (i, 0)),
        out_shape=jax.ShapeDtypeStruct((N, 1), jnp.float32),
    )(s, s)
    return out


# speedup vs baseline: 1.7403x; 1.7403x over previous
import jax
import jax.numpy as jnp
from jax.experimental import pallas as pl

N = 8192
HIDDEN = 32
REG = 1.0
CAPACITY = 3
BLK = 256


def _scores_kernel(xt_ref, w1_ref, b1_ref, w2_ref, b2_ref, s2_ref):
    xt = xt_ref[:, :]
    ht = jnp.maximum(w1_ref[:, :] * xt + b1_ref[:, :], 0.0)
    s = jnp.sum(ht * w2_ref[:, :], axis=0, keepdims=True) + b2_ref[0, 0]
    s2_ref[:, :] = s * (0.5 / REG)


def _softrank_kernel(srow_ref, sfull_ref, out_ref):
    srow = srow_ref[0, :]
    sfull = sfull_ref[0, :]
    t = jnp.tanh(srow[:, None] - sfull[None, :])
    ranks = 0.5 * jnp.sum(t, axis=1) + (0.5 + 0.5 * N)
    other = ranks % CAPACITY
    out = (ranks - other) / CAPACITY + 1.0
    out_ref[:, :] = out.reshape(BLK, 1)


@jax.jit
def kernel(x, W1, b1, W2, b2):
    s2 = pl.pallas_call(
        _scores_kernel,
        out_shape=jax.ShapeDtypeStruct((1, N), jnp.float32),
    )(
        x.reshape(1, N),
        W1.reshape(HIDDEN, 1),
        b1.reshape(HIDDEN, 1),
        W2.reshape(HIDDEN, 1),
        b2.reshape(1, 1),
    )

    out = pl.pallas_call(
        _softrank_kernel,
        grid=(N // BLK,),
        in_specs=[
            pl.BlockSpec((1, BLK), lambda i: (0, i)),
            pl.BlockSpec((1, N), lambda i: (0, 0)),
        ],
        out_specs=pl.BlockSpec((BLK, 1), lambda i: (i, 0)),
        out_shape=jax.ShapeDtypeStruct((N, 1), jnp.float32),
    )(s2, s2)
    return out


# speedup vs baseline: 1.7744x; 1.0196x over previous
import jax
import jax.numpy as jnp
from jax.experimental import pallas as pl

N = 8192
HIDDEN = 32
REG = 1.0
CAPACITY = 3
BLK = 256


def _scores_kernel(xt_ref, w1_ref, b1_ref, w2_ref, b2_ref, s2_ref):
    xt = xt_ref[:, :]
    ht = jnp.maximum(w1_ref[:, :] * xt + b1_ref[:, :], 0.0)
    s = jnp.sum(ht * w2_ref[:, :], axis=0, keepdims=True) + b2_ref[0, 0]
    s2_ref[:, :] = s * (0.5 / REG)


def _softrank_kernel(srow_ref, sfull_ref, out_ref):
    srow = srow_ref[0, :]
    sfull = sfull_ref[0, :]
    t = jnp.tanh(srow[:, None] - sfull[None, :])
    ranks = 0.5 * jnp.sum(t, axis=1) + (0.5 + 0.5 * N)
    other = ranks % CAPACITY
    out = (ranks - other) / CAPACITY + 1.0
    out_ref[:, :] = out.reshape(1, BLK)


@jax.jit
def kernel(x, W1, b1, W2, b2):
    s2 = pl.pallas_call(
        _scores_kernel,
        out_shape=jax.ShapeDtypeStruct((1, N), jnp.float32),
    )(
        x.reshape(1, N),
        W1.reshape(HIDDEN, 1),
        b1.reshape(HIDDEN, 1),
        W2.reshape(HIDDEN, 1),
        b2.reshape(1, 1),
    )

    out = pl.pallas_call(
        _softrank_kernel,
        grid=(N // BLK,),
        in_specs=[
            pl.BlockSpec((1, BLK), lambda i: (0, i)),
            pl.BlockSpec((1, N), lambda i: (0, 0)),
        ],
        out_specs=pl.BlockSpec((1, BLK), lambda i: (0, i)),
        out_shape=jax.ShapeDtypeStruct((1, N), jnp.float32),
    )(s2, s2)
    return out.reshape(N, 1)


# speedup vs baseline: 1.8746x; 1.0565x over previous
import jax
import jax.numpy as jnp
from jax.experimental import pallas as pl

N = 8192
HIDDEN = 32
REG = 1.0
CAPACITY = 3
BLK = 512


def _scores_kernel(xt_ref, w1_ref, b1_ref, w2_ref, b2_ref, s2_ref):
    xt = xt_ref[:, :]
    ht = jnp.maximum(w1_ref[:, :] * xt + b1_ref[:, :], 0.0)
    s = jnp.sum(ht * w2_ref[:, :], axis=0, keepdims=True) + b2_ref[0, 0]
    s2_ref[:, :] = s * (0.5 / REG)


def _softrank_kernel(srow_ref, sfull_ref, out_ref):
    srow = srow_ref[0, :]
    sfull = sfull_ref[0, :]
    t = jnp.tanh(srow[:, None] - sfull[None, :])
    ranks = 0.5 * jnp.sum(t, axis=1) + (0.5 + 0.5 * N)
    other = ranks % CAPACITY
    out = (ranks - other) / CAPACITY + 1.0
    out_ref[:, :] = out.reshape(1, BLK)


@jax.jit
def kernel(x, W1, b1, W2, b2):
    s2 = pl.pallas_call(
        _scores_kernel,
        out_shape=jax.ShapeDtypeStruct((1, N), jnp.float32),
    )(
        x.reshape(1, N),
        W1.reshape(HIDDEN, 1),
        b1.reshape(HIDDEN, 1),
        W2.reshape(HIDDEN, 1),
        b2.reshape(1, 1),
    )

    out = pl.pallas_call(
        _softrank_kernel,
        grid=(N // BLK,),
        in_specs=[
            pl.BlockSpec((1, BLK), lambda i: (0, i)),
            pl.BlockSpec((1, N), lambda i: (0, 0)),
        ],
        out_specs=pl.BlockSpec((1, BLK), lambda i: (0, i)),
        out_shape=jax.ShapeDtypeStruct((1, N), jnp.float32),
    )(s2, s2)
    return out.reshape(N, 1)


# speedup vs baseline: 2.7309x; 1.4568x over previous
import jax
import jax.numpy as jnp
from jax.experimental import pallas as pl
from jax.experimental.pallas import tpu as pltpu

N = 8192
HIDDEN = 32
REG = 1.0
CAPACITY = 3
BLK = 512
B = N // BLK
K = B // 2 + 1


def _tri_kernel(x_ref, w1_ref, b1_ref, w2_ref, b2_ref, out_ref, a_ref, c_ref):
    i = pl.program_id(0)

    @pl.when(i == 0)
    def _init():
        x = x_ref[:, :]
        acc = jnp.full((B, BLK), b2_ref[0, 0], dtype=jnp.float32)
        for k in range(HIDDEN):
            h = jnp.maximum(x * w1_ref[0, k] + b1_ref[0, k], 0.0)
            acc = acc + w2_ref[0, k] * h
        a_ref[:, :] = acc * (0.5 / REG)
        c_ref[:, :] = jnp.zeros((B, BLK), jnp.float32)

    arow = a_ref[pl.ds(i, 1), :][0, :]
    acol_part = arow[:, None]

    acc_t = None
    for k in range(K):
        w = 0.5 if k == K - 1 else 1.0
        jj = jax.lax.rem(i + k, B)
        acol = a_ref[pl.ds(jj, 1), :]
        t = jnp.tanh(acol_part - acol)
        if w != 1.0:
            t = t * w
        acc_t = t if k == 0 else acc_t + t
        if k > 0:
            cs = jnp.sum(t, axis=0).reshape(1, BLK)
            c_ref[pl.ds(jj, 1), :] = c_ref[pl.ds(jj, 1), :] - cs
    u = jnp.sum(acc_t, axis=1).reshape(1, BLK)
    c_ref[pl.ds(i, 1), :] = c_ref[pl.ds(i, 1), :] + u

    @pl.when(i == B - 1)
    def _fin():
        ranks = 0.5 * c_ref[:, :] + (0.5 + 0.5 * N)
        other = ranks % CAPACITY
        out_ref[:, :] = (ranks - other) / CAPACITY + 1.0


@jax.jit
def kernel(x, W1, b1, W2, b2):
    out = pl.pallas_call(
        _tri_kernel,
        grid=(B,),
        in_specs=[
            pl.BlockSpec((B, BLK), lambda i: (0, 0)),
            pl.BlockSpec((1, HIDDEN), lambda i: (0, 0)),
            pl.BlockSpec((1, HIDDEN), lambda i: (0, 0)),
            pl.BlockSpec((1, HIDDEN), lambda i: (0, 0)),
            pl.BlockSpec((1, 1), lambda i: (0, 0)),
        ],
        out_specs=pl.BlockSpec((B, BLK), lambda i: (0, 0)),
        out_shape=jax.ShapeDtypeStruct((B, BLK), jnp.float32),
        scratch_shapes=[
            pltpu.VMEM((B, BLK), jnp.float32),
            pltpu.VMEM((B, BLK), jnp.float32),
        ],
    )(
        x.reshape(B, BLK),
        W1.reshape(1, HIDDEN),
        b1.reshape(1, HIDDEN),
        W2.reshape(1, HIDDEN),
        b2.reshape(1, 1),
    )
    return out.reshape(N, 1)


# speedup vs baseline: 2.8229x; 1.0337x over previous
import jax
import jax.numpy as jnp
from jax.experimental import pallas as pl
from jax.experimental.pallas import tpu as pltpu

N = 8192
HIDDEN = 32
REG = 1.0
CAPACITY = 3
BLK = 512
B = N // BLK
K = B // 2 + 1


def _tri_kernel(x_ref, w1_ref, b1_ref, w2_ref, b2_ref, out_ref, a_ref, c_ref):
    i = pl.program_id(0)

    @pl.when(i == 0)
    def _init():
        x = x_ref[:, :]
        acc = jnp.full((B, BLK), b2_ref[0, 0], dtype=jnp.float32)
        for k in range(HIDDEN):
            h = jnp.maximum(x * w1_ref[0, k] + b1_ref[0, k], 0.0)
            acc = acc + w2_ref[0, k] * h
        a_ref[:, :] = acc * (0.5 / REG)
        c_ref[:, :] = jnp.zeros((B, BLK), jnp.float32)

    arow = a_ref[pl.ds(i, 1), :][0, :]
    acol_part = arow[:, None]

    acc_t = None
    for k in range(K):
        w = 0.5 if k == K - 1 else 1.0
        jj = jax.lax.rem(i + k, B)
        acol = a_ref[pl.ds(jj, 1), :]
        t = jnp.tanh(acol_part - acol)
        if w != 1.0:
            t = t * w
        tf = (t[:, 0:128] + t[:, 128:256]) + (t[:, 256:384] + t[:, 384:512])
        acc_t = tf if k == 0 else acc_t + tf
        if k > 0:
            cs = jnp.sum(t, axis=0).reshape(1, BLK)
            c_ref[pl.ds(jj, 1), :] = c_ref[pl.ds(jj, 1), :] - cs
    u = jnp.sum(acc_t, axis=1).reshape(1, BLK)
    c_ref[pl.ds(i, 1), :] = c_ref[pl.ds(i, 1), :] + u

    @pl.when(i == B - 1)
    def _fin():
        ranks = 0.5 * c_ref[:, :] + (0.5 + 0.5 * N)
        other = ranks % CAPACITY
        out_ref[:, :] = (ranks - other) / CAPACITY + 1.0


@jax.jit
def kernel(x, W1, b1, W2, b2):
    out = pl.pallas_call(
        _tri_kernel,
        grid=(B,),
        in_specs=[
            pl.BlockSpec((B, BLK), lambda i: (0, 0)),
            pl.BlockSpec((1, HIDDEN), lambda i: (0, 0)),
            pl.BlockSpec((1, HIDDEN), lambda i: (0, 0)),
            pl.BlockSpec((1, HIDDEN), lambda i: (0, 0)),
            pl.BlockSpec((1, 1), lambda i: (0, 0)),
        ],
        out_specs=pl.BlockSpec((B, BLK), lambda i: (0, 0)),
        out_shape=jax.ShapeDtypeStruct((B, BLK), jnp.float32),
        scratch_shapes=[
            pltpu.VMEM((B, BLK), jnp.float32),
            pltpu.VMEM((B, BLK), jnp.float32),
        ],
    )(
        x.reshape(B, BLK),
        W1.reshape(1, HIDDEN),
        b1.reshape(1, HIDDEN),
        W2.reshape(1, HIDDEN),
        b2.reshape(1, 1),
    )
    return out.reshape(N, 1)
